# baseline (speedup 1.0000x reference)
import functools

import jax
import jax.numpy as jnp
from jax import lax
from jax.experimental import pallas as pl
from jax.experimental.pallas import tpu as pltpu
from jax.experimental.pallas import tpu_sc as plsc

B = 16384
D = 32
B2 = 2 * B
NW = 32
BPW = B2 // NW
CH = 128
NCH = BPW // CH
TW = 128

ATT = 64
BLK = 1024


def _sc_gather(table128, idx2d):
  mesh = plsc.VectorSubcoreMesh(core_axis_name="c", subcore_axis_name="s")

  @functools.partial(
      pl.kernel,
      mesh=mesh,
      out_type=jax.ShapeDtypeStruct((B2, TW), jnp.float32),
      scratch_types=[
          pltpu.VMEM((NCH, CH), jnp.int32),
          pltpu.VMEM((CH, TW), jnp.float32),
          pltpu.VMEM((CH, TW), jnp.float32),
          pltpu.SemaphoreType.DMA,
          pltpu.SemaphoreType.DMA,
          pltpu.SemaphoreType.DMA,
          pltpu.SemaphoreType.DMA,
      ],
  )
  def k(table_hbm, idx_hbm, out_hbm, idx_v, b0, b1, gs0, gs1, ws0, ws1):
    wid = lax.axis_index("s") * 2 + lax.axis_index("c")
    pltpu.sync_copy(idx_hbm.at[pl.ds(wid * NCH, NCH)], idx_v)
    bufs = (b0, b1)
    gsem = (gs0, gs1)
    wsem = (ws0, ws1)
    gc = [None, None]
    wc = [None, None]
    for j in range(NCH):
      p = j % 2
      if wc[p] is not None:
        wc[p].wait()
      gc[p] = pltpu.async_copy(table_hbm.at[idx_v.at[j]], bufs[p], gsem[p])
      if j >= 1:
        q = (j - 1) % 2
        gc[q].wait()
        wc[q] = pltpu.async_copy(
            bufs[q], out_hbm.at[pl.ds((wid * NCH + j - 1) * CH, CH)], wsem[q])
    q = (NCH - 1) % 2
    gc[q].wait()
    wc[q] = pltpu.async_copy(
        bufs[q], out_hbm.at[pl.ds((wid * NCH + NCH - 1) * CH, CH)], wsem[q])
    wc[0].wait()
    wc[1].wait()

  return k(table128, idx2d)


def _tc_scorer(em2, oh, w1c, b1r, w2, b2r):
  def body(em_ref, oh_ref, w1_ref, b1_ref, w2_ref, b2_ref, o_ref):
    em = em_ref[...]
    oh_ = oh_ref[...]
    parts = []
    for k in range(2):
      acc = em[:, k * TW:k * TW + D] * oh_[:, 4 * k:4 * k + 1]
      for q in range(1, 4):
        acc += em[:, k * TW + q * D:k * TW + (q + 1) * D] * \
            oh_[:, 4 * k + q:4 * k + q + 1]
      parts.append(acc)
    x = jnp.concatenate(parts, axis=1)
    h = jnp.tanh(
        jax.lax.dot_general(
            x, w1_ref[...], (((1,), (0,)), ((), ())),
            preferred_element_type=jnp.float32,
        )
        + b1_ref[...]
    )
    o_ref[...] = (
        jax.lax.dot_general(
            h, w2_ref[...], (((1,), (0,)), ((), ())),
            preferred_element_type=jnp.float32,
        )
        + b2_ref[...]
    )

  return pl.pallas_call(
      body,
      grid=(B // BLK,),
      in_specs=[
          pl.BlockSpec((BLK, 2 * TW), lambda i: (i, 0)),
          pl.BlockSpec((BLK, 8), lambda i: (i, 0)),
          pl.BlockSpec((2 * D, ATT), lambda i: (0, 0)),
          pl.BlockSpec((1, ATT), lambda i: (0, 0)),
          pl.BlockSpec((ATT, 1), lambda i: (0, 0)),
          pl.BlockSpec((1, 1), lambda i: (0, 0)),
      ],
      out_specs=pl.BlockSpec((BLK, 1), lambda i: (i, 0)),
      out_shape=jax.ShapeDtypeStruct((B, 1), jnp.float32),
  )(em2, oh, w1c, b1r, w2, b2r)


def kernel(inds, mask, table, W1, b1, W2, b2):
  table128 = table.reshape(-1, TW)
  idx2d = (inds >> 2).reshape(NW * NCH, CH)
  quad = inds & 3
  oh = jax.nn.one_hot(quad, 4, dtype=jnp.float32).reshape(B, 8)
  em2 = _sc_gather(table128, idx2d).reshape(B, 2 * TW)
  w1c = jnp.concatenate([W1, W1], axis=0) * 0.5
  return _tc_scorer(em2, oh, w1c, b1.reshape(1, ATT), W2, b2.reshape(1, 1))
